# baseline (speedup 1.0000x reference)
import functools

import jax
import jax.numpy as jnp
from jax import lax
from jax.experimental import pallas as pl
from jax.experimental.pallas import tpu as pltpu
from jax.experimental.pallas import tpu_sc as plsc

_BN = 1024
_CH = 2048
_SLAB = 32


def _vq_argmin_body(x_ref, w_ref, sx_ref, sw_ref, idx_ref, loss_ref,
                    m_ref_a, m_ref_b, x2_ref):
    x2_ref[...] = x_ref[...] + x_ref[...]
    sx = sx_ref[...]
    bn = x_ref.shape[0]
    num_k = w_ref.shape[0]
    nch = num_k // _CH
    spc = _CH // _SLAB
    m_bufs = [m_ref_a, m_ref_b]

    sxb = jnp.broadcast_to(sx[None, :], (_SLAB, bn))

    cmin = jnp.full((_SLAB, bn), jnp.inf, jnp.float32)
    cslab = jnp.zeros((_SLAB, bn), jnp.int32)

    def issue_dot(c):
        wc = w_ref[pl.ds(c * _CH, _CH), :]
        m_bufs[c % 2][...] = lax.dot_general(
            wc, x2_ref[...], (((1,), (1,)), ((), ())),
            preferred_element_type=jnp.float32,
        )

    def fold_chunk(c, cmin, cslab):
        for t in range(spc):
            g = c * spc + t
            mv = m_bufs[c % 2][pl.ds(t * _SLAB, _SLAB), :]
            swv = sw_ref[pl.ds(g * _SLAB, _SLAB), :]
            dv = (swv + sxb) - mv
            mask = dv < cmin
            cmin = jnp.where(mask, dv, cmin)
            cslab = jnp.where(mask, g, cslab)
        return cmin, cslab

    issue_dot(0)
    for c in range(1, nch):
        issue_dot(c)
        cmin, cslab = fold_chunk(c - 1, cmin, cslab)
    cmin, cslab = fold_chunk(nch - 1, cmin, cslab)

    riota = lax.broadcasted_iota(jnp.int32, (_SLAB, bn), 0)
    cur_d = cmin
    cur_i = cslab * _SLAB + riota
    rows = _SLAB
    while rows > 1:
        h = rows // 2
        a_d, b_d = cur_d[:h], cur_d[h:rows]
        a_i, b_i = cur_i[:h], cur_i[h:rows]
        better = (b_d < a_d) | ((b_d == a_d) & (b_i < a_i))
        cur_d = jnp.where(better, b_d, a_d)
        cur_i = jnp.where(better, b_i, a_i)
        rows = h
    best_d = cur_d[0]
    best_i = cur_i[0]
    idx_ref[...] = best_i

    @pl.when(pl.program_id(0) == 0)
    def _():
        loss_ref[0, 0] = 0.0

    loss_ref[0, 0] += jnp.sum(best_d)


def _argmin_call(inputs, weight, sx, sw):
    n, d = inputs.shape
    k = weight.shape[0]
    return pl.pallas_call(
        _vq_argmin_body,
        grid=(n // _BN,),
        in_specs=[
            pl.BlockSpec((_BN, d), lambda i: (i, 0)),
            pl.BlockSpec((k, d), lambda i: (0, 0)),
            pl.BlockSpec((_BN,), lambda i: (i,)),
            pl.BlockSpec((k, 1), lambda i: (0, 0)),
        ],
        out_specs=[
            pl.BlockSpec((_BN,), lambda i: (i,)),
            pl.BlockSpec(memory_space=pltpu.SMEM),
        ],
        out_shape=[
            jax.ShapeDtypeStruct((n,), jnp.int32),
            jax.ShapeDtypeStruct((1, 1), jnp.float32),
        ],
        scratch_shapes=[
            pltpu.VMEM((_CH, _BN), jnp.float32),
            pltpu.VMEM((_CH, _BN), jnp.float32),
            pltpu.VMEM((_BN, d), jnp.float32),
        ],
    )(inputs, weight, sx, sw)


def _gather_call(weight, idx):
    n = idx.shape[0]
    k, d = weight.shape
    info = plsc.get_sparse_core_info()
    nc, ns = info.num_cores, info.num_subcores
    nw = nc * ns
    b_per_w = n // nw
    ch = 128
    n_chunks = b_per_w // ch
    mesh = plsc.VectorSubcoreMesh(core_axis_name="c", subcore_axis_name="s")

    @functools.partial(
        pl.kernel, mesh=mesh,
        out_type=jax.ShapeDtypeStruct((n, d), jnp.float32),
        scratch_types=[
            pltpu.VMEM((b_per_w,), jnp.int32),
            pltpu.VMEM((ch, d), jnp.float32),
            pltpu.VMEM((ch, d), jnp.float32),
            pltpu.SemaphoreType.DMA,
            pltpu.SemaphoreType.DMA,
        ],
    )
    def gather_k(table_hbm, idx_hbm, out_hbm, idx_v, rows_a, rows_b, gsem, osem):
        wid = lax.axis_index("s") * nc + lax.axis_index("c")
        base = wid * b_per_w
        bufs = [rows_a, rows_b]
        pltpu.sync_copy(idx_hbm.at[pl.ds(base, b_per_w)], idx_v)
        gh = [None] * n_chunks
        oh = [None] * n_chunks
        gh[0] = pltpu.async_copy(
            table_hbm.at[idx_v.at[pl.ds(0, ch)]], bufs[0], gsem)
        for c in range(n_chunks):
            gh[c].wait()
            if c + 1 < n_chunks:
                if c >= 1:
                    oh[c - 1].wait()
                gh[c + 1] = pltpu.async_copy(
                    table_hbm.at[idx_v.at[pl.ds((c + 1) * ch, ch)]],
                    bufs[(c + 1) % 2], gsem)
            oh[c] = pltpu.async_copy(
                bufs[c % 2], out_hbm.at[pl.ds(base + c * ch, ch)], osem)
        oh[n_chunks - 2].wait()
        oh[n_chunks - 1].wait()

    return gather_k(weight, idx)


def kernel(inputs, weight):
    n, d = inputs.shape
    sx = jnp.sum(inputs * inputs, axis=1)
    sw = jnp.sum(weight * weight, axis=1, keepdims=True)
    idx, loss_sum = _argmin_call(inputs, weight, sx, sw)
    quantized = _gather_call(weight, idx)
    m = loss_sum[0, 0] / jnp.float32(n * d)
    loss = m + jnp.float32(0.25) * m
    encoding_indices = idx.reshape(n, 1, 1)
    return (quantized, loss, encoding_indices)


# speedup vs baseline: 1.0566x; 1.0566x over previous
import functools

import jax
import jax.numpy as jnp
from jax import lax
from jax.experimental import pallas as pl
from jax.experimental.pallas import tpu as pltpu
from jax.experimental.pallas import tpu_sc as plsc

_BN = 1024
_CH = 4096
_SLAB = 8


def _vq_argmin_body(x_ref, w_ref, sx_ref, sw_ref, idx_ref, loss_ref,
                    m_ref_a, m_ref_b, x2_ref):
    x2_ref[...] = x_ref[...] + x_ref[...]
    sx = sx_ref[...]
    bn = x_ref.shape[0]
    num_k = w_ref.shape[0]
    nch = num_k // _CH
    spc = _CH // _SLAB
    m_bufs = [m_ref_a, m_ref_b]

    sxb = jnp.broadcast_to(sx[None, :], (_SLAB, bn))

    cmin = jnp.full((_SLAB, bn), jnp.inf, jnp.float32)
    cslab = jnp.zeros((_SLAB, bn), jnp.int32)

    def issue_dot(c):
        wc = w_ref[pl.ds(c * _CH, _CH), :]
        m_bufs[c % 2][...] = lax.dot_general(
            wc, x2_ref[...], (((1,), (1,)), ((), ())),
            preferred_element_type=jnp.float32,
        )

    def fold_chunk(c, cmin, cslab):
        for t in range(spc):
            g = c * spc + t
            mv = m_bufs[c % 2][pl.ds(t * _SLAB, _SLAB), :]
            swv = sw_ref[pl.ds(g * _SLAB, _SLAB), :]
            dv = (swv + sxb) - mv
            mask = dv < cmin
            cmin = jnp.where(mask, dv, cmin)
            cslab = jnp.where(mask, g, cslab)
        return cmin, cslab

    issue_dot(0)
    for c in range(1, nch):
        issue_dot(c)
        cmin, cslab = fold_chunk(c - 1, cmin, cslab)
    cmin, cslab = fold_chunk(nch - 1, cmin, cslab)

    riota = lax.broadcasted_iota(jnp.int32, (_SLAB, bn), 0)
    cur_d = cmin
    cur_i = cslab * _SLAB + riota
    rows = _SLAB
    while rows > 1:
        h = rows // 2
        a_d, b_d = cur_d[:h], cur_d[h:rows]
        a_i, b_i = cur_i[:h], cur_i[h:rows]
        better = (b_d < a_d) | ((b_d == a_d) & (b_i < a_i))
        cur_d = jnp.where(better, b_d, a_d)
        cur_i = jnp.where(better, b_i, a_i)
        rows = h
    best_d = cur_d[0]
    best_i = cur_i[0]
    idx_ref[...] = best_i

    @pl.when(pl.program_id(0) == 0)
    def _():
        loss_ref[0, 0] = 0.0

    loss_ref[0, 0] += jnp.sum(best_d)


def _argmin_call(inputs, weight, sx, sw):
    n, d = inputs.shape
    k = weight.shape[0]
    return pl.pallas_call(
        _vq_argmin_body,
        grid=(n // _BN,),
        in_specs=[
            pl.BlockSpec((_BN, d), lambda i: (i, 0)),
            pl.BlockSpec((k, d), lambda i: (0, 0)),
            pl.BlockSpec((_BN,), lambda i: (i,)),
            pl.BlockSpec((k, 1), lambda i: (0, 0)),
        ],
        out_specs=[
            pl.BlockSpec((_BN,), lambda i: (i,)),
            pl.BlockSpec(memory_space=pltpu.SMEM),
        ],
        out_shape=[
            jax.ShapeDtypeStruct((n,), jnp.int32),
            jax.ShapeDtypeStruct((1, 1), jnp.float32),
        ],
        scratch_shapes=[
            pltpu.VMEM((_CH, _BN), jnp.float32),
            pltpu.VMEM((_CH, _BN), jnp.float32),
            pltpu.VMEM((_BN, d), jnp.float32),
        ],
    )(inputs, weight, sx, sw)


def _gather_call(weight, idx):
    n = idx.shape[0]
    k, d = weight.shape
    info = plsc.get_sparse_core_info()
    nc, ns = info.num_cores, info.num_subcores
    nw = nc * ns
    b_per_w = n // nw
    ch = 128
    n_chunks = b_per_w // ch
    mesh = plsc.VectorSubcoreMesh(core_axis_name="c", subcore_axis_name="s")

    @functools.partial(
        pl.kernel, mesh=mesh,
        out_type=jax.ShapeDtypeStruct((n, d), jnp.float32),
        scratch_types=[
            pltpu.VMEM((b_per_w,), jnp.int32),
            pltpu.VMEM((ch, d), jnp.float32),
            pltpu.VMEM((ch, d), jnp.float32),
            pltpu.SemaphoreType.DMA,
            pltpu.SemaphoreType.DMA,
        ],
    )
    def gather_k(table_hbm, idx_hbm, out_hbm, idx_v, rows_a, rows_b, gsem, osem):
        wid = lax.axis_index("s") * nc + lax.axis_index("c")
        base = wid * b_per_w
        bufs = [rows_a, rows_b]
        pltpu.sync_copy(idx_hbm.at[pl.ds(base, b_per_w)], idx_v)
        gh = [None] * n_chunks
        oh = [None] * n_chunks
        gh[0] = pltpu.async_copy(
            table_hbm.at[idx_v.at[pl.ds(0, ch)]], bufs[0], gsem)
        for c in range(n_chunks):
            gh[c].wait()
            if c + 1 < n_chunks:
                if c >= 1:
                    oh[c - 1].wait()
                gh[c + 1] = pltpu.async_copy(
                    table_hbm.at[idx_v.at[pl.ds((c + 1) * ch, ch)]],
                    bufs[(c + 1) % 2], gsem)
            oh[c] = pltpu.async_copy(
                bufs[c % 2], out_hbm.at[pl.ds(base + c * ch, ch)], osem)
        oh[n_chunks - 2].wait()
        oh[n_chunks - 1].wait()

    return gather_k(weight, idx)


def kernel(inputs, weight):
    n, d = inputs.shape
    sx = jnp.sum(inputs * inputs, axis=1)
    sw = jnp.sum(weight * weight, axis=1, keepdims=True)
    idx, loss_sum = _argmin_call(inputs, weight, sx, sw)
    quantized = _gather_call(weight, idx)
    m = loss_sum[0, 0] / jnp.float32(n * d)
    loss = m + jnp.float32(0.25) * m
    encoding_indices = idx.reshape(n, 1, 1)
    return (quantized, loss, encoding_indices)
